# baseline (speedup 1.0000x reference)
import functools

import jax
import jax.numpy as jnp
from jax import lax
from jax.experimental import pallas as pl
from jax.experimental.pallas import tpu as pltpu
from jax.experimental.pallas import tpu_sc as plsc

N = 32768
D = 128
NSEG = 64
L = 16
NC, NS = 2, 16
NW = NC * NS
ROWS_W = N // NW
R = 256
NCH = ROWS_W // R
GPC = R // L
CPD = D // L
EPS = 1e-06
LAMBDA_PCSL = 0.1
LAMBDA_GPAL = 0.1


def _tree_sum(vs):
    while len(vs) > 1:
        vs = [a + b for a, b in zip(vs[::2], vs[1::2])]
    return vs[0]


def _sc_partials(features, patient_ids):
    mesh = plsc.VectorSubcoreMesh(core_axis_name="c", subcore_axis_name="s")

    @functools.partial(
        pl.kernel,
        out_type=[
            jax.ShapeDtypeStruct((NW, NSEG, D), jnp.float32),
            jax.ShapeDtypeStruct((NW, L), jnp.float32),
        ],
        mesh=mesh,
        compiler_params=pltpu.CompilerParams(needs_layout_passes=False),
        scratch_types=[
            pltpu.VMEM((ROWS_W,), jnp.int32),
            pltpu.VMEM((R, D), jnp.float32),
            pltpu.VMEM((R, D), jnp.float32),
            pltpu.VMEM((NSEG, D), jnp.float32),
            pltpu.VMEM((L,), jnp.float32),
            pltpu.SemaphoreType.DMA,
            pltpu.SemaphoreType.DMA,
        ],
    )
    def k(feat_hbm, ids_hbm, psum_hbm, psq_hbm, ids_v, buf0_v, buf1_v,
          acc_v, sqout_v, sem0, sem1):
        wid = lax.axis_index("s") * NC + lax.axis_index("c")
        base = wid * ROWS_W
        bufs = [buf0_v, buf1_v]
        sems = [sem0, sem1]

        handles = {}
        for ch in range(min(2, NCH)):
            handles[ch] = pltpu.async_copy(
                feat_hbm.at[pl.ds(base + ch * R, R), :], bufs[ch % 2],
                sems[ch % 2])
        pltpu.sync_copy(ids_hbm.at[pl.ds(base, ROWS_W)], ids_v)

        zeros = jnp.zeros((L,), jnp.float32)

        def zacc(i, _):
            for c in range(CPD):
                acc_v[i, pl.ds(c * L, L)] = zeros
            return 0

        lax.fori_loop(0, NSEG, zacc, 0)

        iota = lax.iota(jnp.int32, L)
        sqs = tuple(zeros for _ in range(CPD))

        for ch in range(NCH):
            buf_v = bufs[ch % 2]
            handles[ch].wait()

            def grp(g, sqs):
                rb = g * L
                v = ids_v[pl.ds(ch * R + g * L, L)]
                lo = jnp.min(v)
                hi = jnp.max(v)

                def fast(sqs):
                    new = []
                    for c in range(CPD):
                        data = [buf_v[rb + r, pl.ds(c * L, L)]
                                for r in range(L)]
                        s = _tree_sum(data)
                        sq = _tree_sum([x * x for x in data])
                        plsc.addupdate(acc_v.at[lo, pl.ds(c * L, L)], s)
                        new.append(sqs[c] + sq)
                    return tuple(new)

                def slow(sqs):
                    sqs = list(sqs)
                    for r in range(L):
                        seg = jnp.sum(jnp.where(iota == r, v, 0))
                        for c in range(CPD):
                            data = buf_v[rb + r, pl.ds(c * L, L)]
                            plsc.addupdate(
                                acc_v.at[seg, pl.ds(c * L, L)], data)
                            sqs[c] = sqs[c] + data * data
                    return tuple(sqs)

                return lax.cond(lo == hi, fast, slow, sqs)

            sqs = plsc.parallel_loop(0, GPC, carry=sqs)(grp)
            if ch + 2 < NCH:
                handles[ch + 2] = pltpu.async_copy(
                    feat_hbm.at[pl.ds(base + (ch + 2) * R, R), :], buf_v,
                    sems[ch % 2])

        sqout_v[...] = _tree_sum(list(sqs))
        pltpu.sync_copy(acc_v, psum_hbm.at[wid])
        pltpu.sync_copy(sqout_v, psq_hbm.at[wid])

    return k(features, patient_ids)


def _epi_body(ps_ref, sq_ref, ids_ref, out_ref):
    sums = ps_ref[0]
    for t in range(1, NW):
        sums = sums + ps_ref[t]
    total_sq = jnp.sum(sq_ref[...])

    seg_iota = lax.broadcasted_iota(jnp.int32, (NSEG, D), 0)

    def cnt_body(r, cm):
        row = ids_ref[pl.ds(r, 1), :]
        m = jnp.broadcast_to(row, (NSEG, D)) == seg_iota
        return cm + m.astype(jnp.float32)

    cnt = lax.fori_loop(0, N // D, cnt_body,
                        jnp.zeros((NSEG, D), jnp.float32))
    cnt = jnp.broadcast_to(jnp.sum(cnt, axis=1, keepdims=True), (NSEG, D))

    safe = jnp.maximum(cnt, 1.0)
    cent = sums / safe
    csq = cent * cent
    within = total_sq - jnp.sum(cnt * csq)
    validf = (cnt > 0).astype(jnp.float32)
    kseg = jnp.sum(validf) / D
    csqsum = jnp.sum(validf * csq)
    svec = jnp.sum(validf * cent, axis=0, keepdims=True)
    ssq = jnp.sum(svec * svec)
    between = kseg * csqsum - ssq
    loss_pcsl = within / (between + EPS)
    loss_gpal = (csqsum - ssq / kseg) / kseg
    loss = LAMBDA_PCSL * loss_pcsl + LAMBDA_GPAL * loss_gpal
    out_ref[...] = jnp.broadcast_to(loss, (1, 1))


def kernel(features, patient_ids):
    psum, psq = _sc_partials(features, patient_ids)
    ids2d = patient_ids.reshape(N // D, D)
    out = pl.pallas_call(
        _epi_body,
        out_shape=jax.ShapeDtypeStruct((1, 1), jnp.float32),
    )(psum, psq, ids2d)
    return out[0, 0]


# speedup vs baseline: 1.1000x; 1.1000x over previous
import functools

import jax
import jax.numpy as jnp
from jax import lax
from jax.experimental import pallas as pl
from jax.experimental.pallas import tpu as pltpu
from jax.experimental.pallas import tpu_sc as plsc

N = 32768
D = 128
NSEG = 64
L = 16
NC, NS = 2, 16
NW = NC * NS
ROWS_W = N // NW
R = 256
NCH = ROWS_W // R
GPC = R // L
CPD = D // L
EPS = 1e-06
LAMBDA_PCSL = 0.1
LAMBDA_GPAL = 0.1


def _tree_sum(vs):
    while len(vs) > 1:
        vs = [a + b for a, b in zip(vs[::2], vs[1::2])]
    return vs[0]


def _sc_partials(features, patient_ids):
    mesh = plsc.VectorSubcoreMesh(core_axis_name="c", subcore_axis_name="s")

    @functools.partial(
        pl.kernel,
        out_type=[
            jax.ShapeDtypeStruct((NW, NSEG, D), jnp.float32),
            jax.ShapeDtypeStruct((NW, L), jnp.float32),
        ],
        mesh=mesh,
        compiler_params=pltpu.CompilerParams(needs_layout_passes=False),
        scratch_types=[
            pltpu.VMEM((ROWS_W,), jnp.int32),
            pltpu.VMEM((R, D), jnp.float32),
            pltpu.VMEM((R, D), jnp.float32),
            pltpu.VMEM((NSEG, D), jnp.float32),
            pltpu.VMEM((L,), jnp.float32),
            pltpu.SemaphoreType.DMA,
            pltpu.SemaphoreType.DMA,
        ],
    )
    def k(feat_hbm, ids_hbm, psum_hbm, psq_hbm, ids_v, buf0_v, buf1_v,
          acc_v, sqout_v, sem0, sem1):
        wid = lax.axis_index("s") * NC + lax.axis_index("c")
        base = wid * ROWS_W
        bufs = [buf0_v, buf1_v]
        sems = [sem0, sem1]

        handles = {}
        for ch in range(min(2, NCH)):
            handles[ch] = pltpu.async_copy(
                feat_hbm.at[pl.ds(base + ch * R, R), :], bufs[ch % 2],
                sems[ch % 2])
        pltpu.sync_copy(ids_hbm.at[pl.ds(base, ROWS_W)], ids_v)

        zeros = jnp.zeros((L,), jnp.float32)

        def zacc(i, _):
            for c in range(CPD):
                acc_v[i, pl.ds(c * L, L)] = zeros
            return 0

        lax.fori_loop(0, NSEG, zacc, 0)

        iota = lax.iota(jnp.int32, L)
        sqs = tuple(zeros for _ in range(CPD))

        for ch in range(NCH):
            buf_v = bufs[ch % 2]
            handles[ch].wait()

            def grp(g, sqs):
                rb = g * L
                v = ids_v[pl.ds(ch * R + g * L, L)]
                lo = jnp.min(v)
                hi = jnp.max(v)

                def fast(sqs):
                    ssum = [None] * CPD
                    ssq = [None] * CPD
                    for r in range(L):
                        for c in range(CPD):
                            x = buf_v[rb + r, pl.ds(c * L, L)]
                            x2 = x * x
                            if r == 0:
                                ssum[c] = x
                                ssq[c] = x2
                            else:
                                ssum[c] = ssum[c] + x
                                ssq[c] = ssq[c] + x2
                    new = []
                    for c in range(CPD):
                        plsc.addupdate(acc_v.at[lo, pl.ds(c * L, L)],
                                       ssum[c])
                        new.append(sqs[c] + ssq[c])
                    return tuple(new)

                def slow(sqs):
                    sqs = list(sqs)
                    for r in range(L):
                        seg = jnp.sum(jnp.where(iota == r, v, 0))
                        for c in range(CPD):
                            data = buf_v[rb + r, pl.ds(c * L, L)]
                            plsc.addupdate(
                                acc_v.at[seg, pl.ds(c * L, L)], data)
                            sqs[c] = sqs[c] + data * data
                    return tuple(sqs)

                return lax.cond(lo == hi, fast, slow, sqs)

            sqs = plsc.parallel_loop(0, GPC, carry=sqs)(grp)
            if ch + 2 < NCH:
                handles[ch + 2] = pltpu.async_copy(
                    feat_hbm.at[pl.ds(base + (ch + 2) * R, R), :], buf_v,
                    sems[ch % 2])

        sqout_v[...] = _tree_sum(list(sqs))
        pltpu.sync_copy(acc_v, psum_hbm.at[wid])
        pltpu.sync_copy(sqout_v, psq_hbm.at[wid])

    return k(features, patient_ids)


def _epi_body(ps_ref, sq_ref, ids_ref, out_ref):
    sums = ps_ref[0]
    for t in range(1, NW):
        sums = sums + ps_ref[t]
    total_sq = jnp.sum(sq_ref[...])

    seg_iota = lax.broadcasted_iota(jnp.int32, (NSEG, D), 0)

    def cnt_body(r, cm):
        row = ids_ref[pl.ds(r, 1), :]
        m = jnp.broadcast_to(row, (NSEG, D)) == seg_iota
        return cm + m.astype(jnp.float32)

    cnt = lax.fori_loop(0, N // D, cnt_body,
                        jnp.zeros((NSEG, D), jnp.float32))
    cnt = jnp.broadcast_to(jnp.sum(cnt, axis=1, keepdims=True), (NSEG, D))

    safe = jnp.maximum(cnt, 1.0)
    cent = sums / safe
    csq = cent * cent
    within = total_sq - jnp.sum(cnt * csq)
    validf = (cnt > 0).astype(jnp.float32)
    kseg = jnp.sum(validf) / D
    csqsum = jnp.sum(validf * csq)
    svec = jnp.sum(validf * cent, axis=0, keepdims=True)
    ssq = jnp.sum(svec * svec)
    between = kseg * csqsum - ssq
    loss_pcsl = within / (between + EPS)
    loss_gpal = (csqsum - ssq / kseg) / kseg
    loss = LAMBDA_PCSL * loss_pcsl + LAMBDA_GPAL * loss_gpal
    out_ref[...] = jnp.broadcast_to(loss, (1, 1))


def kernel(features, patient_ids):
    psum, psq = _sc_partials(features, patient_ids)
    ids2d = patient_ids.reshape(N // D, D)
    out = pl.pallas_call(
        _epi_body,
        out_shape=jax.ShapeDtypeStruct((1, 1), jnp.float32),
    )(psum, psq, ids2d)
    return out[0, 0]


# speedup vs baseline: 1.2520x; 1.1381x over previous
import functools

import jax
import jax.numpy as jnp
from jax import lax
from jax.experimental import pallas as pl
from jax.experimental.pallas import tpu as pltpu
from jax.experimental.pallas import tpu_sc as plsc

N = 32768
D = 128
NSEG = 64
L = 16
NC, NS = 2, 16
NW = NC * NS
ROWS_W = N // NW
R = 256
NCH = ROWS_W // R
GPC = R // L
CPD = D // L
EPS = 1e-06
LAMBDA_PCSL = 0.1
LAMBDA_GPAL = 0.1


def _tree_sum(vs):
    while len(vs) > 1:
        vs = [a + b for a, b in zip(vs[::2], vs[1::2])]
    return vs[0]


def _sc_partials(features, patient_ids):
    mesh = plsc.VectorSubcoreMesh(core_axis_name="c", subcore_axis_name="s")

    @functools.partial(
        pl.kernel,
        out_type=[
            jax.ShapeDtypeStruct((NW, NSEG, D), jnp.float32),
            jax.ShapeDtypeStruct((NW, L), jnp.float32),
            jax.ShapeDtypeStruct((NW, NSEG, L), jnp.float32),
        ],
        mesh=mesh,
        compiler_params=pltpu.CompilerParams(needs_layout_passes=False),
        scratch_types=[
            pltpu.VMEM((ROWS_W,), jnp.int32),
            pltpu.VMEM((R, D), jnp.float32),
            pltpu.VMEM((R, D), jnp.float32),
            pltpu.VMEM((NSEG, D), jnp.float32),
            pltpu.VMEM((NSEG, L), jnp.float32),
            pltpu.VMEM((L,), jnp.float32),
            pltpu.SemaphoreType.DMA,
            pltpu.SemaphoreType.DMA,
        ],
    )
    def k(feat_hbm, ids_hbm, psum_hbm, psq_hbm, pcnt_hbm, ids_v, buf0_v,
          buf1_v, acc_v, cnt_v, sqout_v, sem0, sem1):
        wid = lax.axis_index("s") * NC + lax.axis_index("c")
        base = wid * ROWS_W
        bufs = [buf0_v, buf1_v]
        sems = [sem0, sem1]

        handles = {}
        for ch in range(min(2, NCH)):
            handles[ch] = pltpu.async_copy(
                feat_hbm.at[pl.ds(base + ch * R, R), :], bufs[ch % 2],
                sems[ch % 2])
        pltpu.sync_copy(ids_hbm.at[pl.ds(base, ROWS_W)], ids_v)

        zeros = jnp.zeros((L,), jnp.float32)

        def zacc(i, _):
            for c in range(CPD):
                acc_v[i, pl.ds(c * L, L)] = zeros
            cnt_v[i, :] = zeros
            return 0

        lax.fori_loop(0, NSEG, zacc, 0)

        iota = lax.iota(jnp.int32, L)
        sqs = tuple(zeros for _ in range(CPD))

        for ch in range(NCH):
            buf_v = bufs[ch % 2]
            handles[ch].wait()

            def grp(g, sqs):
                rb = g * L
                v = ids_v[pl.ds(ch * R + g * L, L)]
                lo = jnp.min(v)
                hi = jnp.max(v)

                def fast(sqs):
                    ssum = [None] * CPD
                    ssq = [None] * CPD
                    for r in range(L):
                        for c in range(CPD):
                            x = buf_v[rb + r, pl.ds(c * L, L)]
                            x2 = x * x
                            if r == 0:
                                ssum[c] = x
                                ssq[c] = x2
                            else:
                                ssum[c] = ssum[c] + x
                                ssq[c] = ssq[c] + x2
                    new = []
                    for c in range(CPD):
                        plsc.addupdate(acc_v.at[lo, pl.ds(c * L, L)],
                                       ssum[c])
                        new.append(sqs[c] + ssq[c])
                    plsc.addupdate(cnt_v.at[lo], jnp.full((L,), 16.0,
                                                          jnp.float32))
                    return tuple(new)

                def slow(sqs):
                    ones = jnp.full((L,), 1.0, jnp.float32)

                    def row_body(r, sqs):
                        seg = jnp.sum(jnp.where(iota == r, v, 0))
                        new = []
                        for c in range(CPD):
                            data = buf_v[rb + r, pl.ds(c * L, L)]
                            plsc.addupdate(
                                acc_v.at[seg, pl.ds(c * L, L)], data)
                            new.append(sqs[c] + data * data)
                        plsc.addupdate(cnt_v.at[seg], ones)
                        return tuple(new)

                    return lax.fori_loop(0, L, row_body, sqs)

                return lax.cond(lo == hi, fast, slow, sqs)

            sqs = plsc.parallel_loop(0, GPC, carry=sqs)(grp)
            if ch + 2 < NCH:
                handles[ch + 2] = pltpu.async_copy(
                    feat_hbm.at[pl.ds(base + (ch + 2) * R, R), :], buf_v,
                    sems[ch % 2])

        sqout_v[...] = _tree_sum(list(sqs))
        pltpu.sync_copy(acc_v, psum_hbm.at[wid])
        pltpu.sync_copy(sqout_v, psq_hbm.at[wid])
        pltpu.sync_copy(cnt_v, pcnt_hbm.at[wid])

    return k(features, patient_ids)


def _epi_body(ps_ref, sq_ref, pc_ref, out_ref):
    sums = ps_ref[0]
    cnt_l = pc_ref[0]
    for t in range(1, NW):
        sums = sums + ps_ref[t]
        cnt_l = cnt_l + pc_ref[t]
    total_sq = jnp.sum(sq_ref[...])
    cnt = jnp.broadcast_to(cnt_l[:, 0:1], (NSEG, D))

    safe = jnp.maximum(cnt, 1.0)
    cent = sums / safe
    csq = cent * cent
    within = total_sq - jnp.sum(cnt * csq)
    validf = (cnt > 0).astype(jnp.float32)
    kseg = jnp.sum(validf) / D
    csqsum = jnp.sum(validf * csq)
    svec = jnp.sum(validf * cent, axis=0, keepdims=True)
    ssq = jnp.sum(svec * svec)
    between = kseg * csqsum - ssq
    loss_pcsl = within / (between + EPS)
    loss_gpal = (csqsum - ssq / kseg) / kseg
    loss = LAMBDA_PCSL * loss_pcsl + LAMBDA_GPAL * loss_gpal
    out_ref[...] = jnp.broadcast_to(loss, (1, 1))


def kernel(features, patient_ids):
    psum, psq, pcnt = _sc_partials(features, patient_ids)
    out = pl.pallas_call(
        _epi_body,
        out_shape=jax.ShapeDtypeStruct((1, 1), jnp.float32),
    )(psum, psq, pcnt)
    return out[0, 0]


# speedup vs baseline: 1.2627x; 1.0086x over previous
import functools

import jax
import jax.numpy as jnp
from jax import lax
from jax.experimental import pallas as pl
from jax.experimental.pallas import tpu as pltpu
from jax.experimental.pallas import tpu_sc as plsc

N = 32768
D = 128
NSEG = 64
L = 16
NC, NS = 2, 16
NW = NC * NS
ROWS_W = N // NW
R = 256
NCH = ROWS_W // R
GPC = R // L
CPD = D // L
EPS = 1e-06
LAMBDA_PCSL = 0.1
LAMBDA_GPAL = 0.1


def _tree_sum(vs):
    while len(vs) > 1:
        vs = [a + b for a, b in zip(vs[::2], vs[1::2])]
    return vs[0]


def _sc_partials(features, patient_ids):
    mesh = plsc.VectorSubcoreMesh(core_axis_name="c", subcore_axis_name="s")

    @functools.partial(
        pl.kernel,
        out_type=[
            jax.ShapeDtypeStruct((NW, NSEG, D), jnp.float32),
            jax.ShapeDtypeStruct((NW, L), jnp.float32),
            jax.ShapeDtypeStruct((NW, NSEG, L), jnp.float32),
        ],
        mesh=mesh,
        compiler_params=pltpu.CompilerParams(needs_layout_passes=False),
        scratch_types=[
            pltpu.VMEM((ROWS_W,), jnp.int32),
            pltpu.VMEM((R, D), jnp.float32),
            pltpu.VMEM((R, D), jnp.float32),
            pltpu.VMEM((NSEG, D), jnp.float32),
            pltpu.VMEM((NSEG, L), jnp.float32),
            pltpu.VMEM((L,), jnp.float32),
            pltpu.SemaphoreType.DMA,
            pltpu.SemaphoreType.DMA,
        ],
    )
    def k(feat_hbm, ids_hbm, psum_hbm, psq_hbm, pcnt_hbm, ids_v, buf0_v,
          buf1_v, acc_v, cnt_v, sqout_v, sem0, sem1):
        wid = lax.axis_index("s") * NC + lax.axis_index("c")
        base = wid * ROWS_W
        bufs = [buf0_v, buf1_v]
        sems = [sem0, sem1]

        handles = {}
        for ch in range(min(2, NCH)):
            handles[ch] = pltpu.async_copy(
                feat_hbm.at[pl.ds(base + ch * R, R), :], bufs[ch % 2],
                sems[ch % 2])
        pltpu.sync_copy(ids_hbm.at[pl.ds(base, ROWS_W)], ids_v)

        zeros = jnp.zeros((L,), jnp.float32)

        def zacc(i, _):
            for c in range(CPD):
                acc_v[i, pl.ds(c * L, L)] = zeros
            cnt_v[i, :] = zeros
            return 0

        lax.fori_loop(0, NSEG, zacc, 0)

        iota = lax.iota(jnp.int32, L)
        sqs = tuple(zeros for _ in range(CPD))

        for ch in range(NCH):
            buf_v = bufs[ch % 2]
            handles[ch].wait()

            def grp(g, sqs):
                rb = g * L
                off = ch * R + g * L
                v = ids_v[pl.ds(off, L)]
                lo = v[0]
                hi = v[L - 1]

                def fast(sqs):
                    ssum = [None] * CPD
                    ssq = [None] * CPD
                    for r in range(L):
                        for c in range(CPD):
                            x = buf_v[rb + r, pl.ds(c * L, L)]
                            x2 = x * x
                            if r == 0:
                                ssum[c] = x
                                ssq[c] = x2
                            else:
                                ssum[c] = ssum[c] + x
                                ssq[c] = ssq[c] + x2
                    new = []
                    for c in range(CPD):
                        plsc.addupdate(acc_v.at[lo, pl.ds(c * L, L)],
                                       ssum[c])
                        new.append(sqs[c] + ssq[c])
                    plsc.addupdate(cnt_v.at[lo], jnp.full((L,), 16.0,
                                                          jnp.float32))
                    return tuple(new)

                def slow(sqs):
                    ones = jnp.full((L,), 1.0, jnp.float32)

                    def row_body(r, sqs):
                        seg = jnp.sum(jnp.where(iota == r, v, 0))
                        new = []
                        for c in range(CPD):
                            data = buf_v[rb + r, pl.ds(c * L, L)]
                            plsc.addupdate(
                                acc_v.at[seg, pl.ds(c * L, L)], data)
                            new.append(sqs[c] + data * data)
                        plsc.addupdate(cnt_v.at[seg], ones)
                        return tuple(new)

                    return lax.fori_loop(0, L, row_body, sqs)

                return lax.cond(lo == hi, fast, slow, sqs)

            sqs = plsc.parallel_loop(0, GPC, carry=sqs)(grp)
            if ch + 2 < NCH:
                handles[ch + 2] = pltpu.async_copy(
                    feat_hbm.at[pl.ds(base + (ch + 2) * R, R), :], buf_v,
                    sems[ch % 2])

        sqout_v[...] = _tree_sum(list(sqs))
        pltpu.sync_copy(acc_v, psum_hbm.at[wid])
        pltpu.sync_copy(sqout_v, psq_hbm.at[wid])
        pltpu.sync_copy(cnt_v, pcnt_hbm.at[wid])

    return k(features, patient_ids)


def _epi_body(ps_ref, sq_ref, pc_ref, out_ref):
    sums = ps_ref[0]
    cnt_l = pc_ref[0]
    for t in range(1, NW):
        sums = sums + ps_ref[t]
        cnt_l = cnt_l + pc_ref[t]
    total_sq = jnp.sum(sq_ref[...])
    cnt = jnp.broadcast_to(cnt_l[:, 0:1], (NSEG, D))

    safe = jnp.maximum(cnt, 1.0)
    cent = sums / safe
    csq = cent * cent
    within = total_sq - jnp.sum(cnt * csq)
    validf = (cnt > 0).astype(jnp.float32)
    kseg = jnp.sum(validf) / D
    csqsum = jnp.sum(validf * csq)
    svec = jnp.sum(validf * cent, axis=0, keepdims=True)
    ssq = jnp.sum(svec * svec)
    between = kseg * csqsum - ssq
    loss_pcsl = within / (between + EPS)
    loss_gpal = (csqsum - ssq / kseg) / kseg
    loss = LAMBDA_PCSL * loss_pcsl + LAMBDA_GPAL * loss_gpal
    out_ref[...] = jnp.broadcast_to(loss, (1, 1))


def kernel(features, patient_ids):
    psum, psq, pcnt = _sc_partials(features, patient_ids)
    out = pl.pallas_call(
        _epi_body,
        out_shape=jax.ShapeDtypeStruct((1, 1), jnp.float32),
    )(psum, psq, pcnt)
    return out[0, 0]


# speedup vs baseline: 1.3358x; 1.0579x over previous
import functools

import jax
import jax.numpy as jnp
from jax import lax
from jax.experimental import pallas as pl
from jax.experimental.pallas import tpu as pltpu
from jax.experimental.pallas import tpu_sc as plsc

N = 32768
D = 128
NSEG = 64
L = 16
NC, NS = 2, 16
NW = NC * NS
ROWS_W = N // NW
R = 256
NCH = ROWS_W // R
GPC = R // L
CPD = D // L
EPS = 1e-06
LAMBDA_PCSL = 0.1
LAMBDA_GPAL = 0.1


def _tree_sum(vs):
    while len(vs) > 1:
        vs = [a + b for a, b in zip(vs[::2], vs[1::2])]
    return vs[0]


def _sc_partials(features, patient_ids):
    mesh = plsc.VectorSubcoreMesh(core_axis_name="c", subcore_axis_name="s")

    @functools.partial(
        pl.kernel,
        out_type=[
            jax.ShapeDtypeStruct((NW, NSEG, D), jnp.float32),
            jax.ShapeDtypeStruct((NW, L), jnp.float32),
            jax.ShapeDtypeStruct((NW, NSEG, L), jnp.float32),
        ],
        mesh=mesh,
        compiler_params=pltpu.CompilerParams(needs_layout_passes=False),
        scratch_types=[
            pltpu.VMEM((ROWS_W,), jnp.int32),
            pltpu.VMEM((2 * R, D), jnp.float32),
            pltpu.VMEM((NSEG, D), jnp.float32),
            pltpu.VMEM((NSEG, L), jnp.float32),
            pltpu.VMEM((L,), jnp.float32),
            pltpu.SemaphoreType.DMA,
            pltpu.SemaphoreType.DMA,
        ],
    )
    def k(feat_hbm, ids_hbm, psum_hbm, psq_hbm, pcnt_hbm, ids_v, buf_v,
          acc_v, cnt_v, sqout_v, sem0, sem1):
        wid = lax.axis_index("s") * NC + lax.axis_index("c")
        base = wid * ROWS_W
        sems = [sem0, sem1]
        halves = [buf_v.at[pl.ds(0, R), :], buf_v.at[pl.ds(R, R), :]]

        for ch in range(min(2, NCH)):
            pltpu.async_copy(
                feat_hbm.at[pl.ds(base + ch * R, R), :], halves[ch % 2],
                sems[ch % 2])
        pltpu.sync_copy(ids_hbm.at[pl.ds(base, ROWS_W)], ids_v)

        zeros = jnp.zeros((L,), jnp.float32)

        def zacc(i, _):
            for c in range(CPD):
                acc_v[i, pl.ds(c * L, L)] = zeros
            cnt_v[i, :] = zeros
            return 0

        lax.fori_loop(0, NSEG, zacc, 0)

        iota = lax.iota(jnp.int32, L)
        sqs = tuple(zeros for _ in range(CPD))

        def chunk_body(ch, sqs):
            slot = lax.rem(ch, 2)
            for p in range(2):
                @pl.when(slot == p)
                def _():
                    pltpu.make_async_copy(
                        feat_hbm.at[pl.ds(base, R), :], halves[p],
                        sems[p]).wait()
            rb0 = slot * R

            def grp(g, sqs):
                rb = rb0 + g * L
                off = ch * R + g * L
                v = ids_v[pl.ds(off, L)]
                lo = v[0]
                hi = v[L - 1]

                def fast(sqs):
                    ssum = [None] * CPD
                    ssq = [None] * CPD
                    for r in range(L):
                        for c in range(CPD):
                            x = buf_v[rb + r, pl.ds(c * L, L)]
                            x2 = x * x
                            if r == 0:
                                ssum[c] = x
                                ssq[c] = x2
                            else:
                                ssum[c] = ssum[c] + x
                                ssq[c] = ssq[c] + x2
                    new = []
                    for c in range(CPD):
                        plsc.addupdate(acc_v.at[lo, pl.ds(c * L, L)],
                                       ssum[c])
                        new.append(sqs[c] + ssq[c])
                    plsc.addupdate(cnt_v.at[lo], jnp.full((L,), 16.0,
                                                          jnp.float32))
                    return tuple(new)

                def slow(sqs):
                    ones = jnp.full((L,), 1.0, jnp.float32)

                    def row_body(r, sqs):
                        seg = jnp.sum(jnp.where(iota == r, v, 0))
                        new = []
                        for c in range(CPD):
                            data = buf_v[rb + r, pl.ds(c * L, L)]
                            plsc.addupdate(
                                acc_v.at[seg, pl.ds(c * L, L)], data)
                            new.append(sqs[c] + data * data)
                        plsc.addupdate(cnt_v.at[seg], ones)
                        return tuple(new)

                    return lax.fori_loop(0, L, row_body, sqs)

                return lax.cond(lo == hi, fast, slow, sqs)

            sqs = plsc.parallel_loop(0, GPC, carry=sqs)(grp)
            for p in range(2):
                @pl.when((slot == p) & (ch + 2 < NCH))
                def _():
                    pltpu.async_copy(
                        feat_hbm.at[pl.ds(base + (ch + 2) * R, R), :],
                        halves[p], sems[p])
            return sqs

        sqs = lax.fori_loop(0, NCH, chunk_body, sqs)

        sqout_v[...] = _tree_sum(list(sqs))
        pltpu.sync_copy(acc_v, psum_hbm.at[wid])
        pltpu.sync_copy(sqout_v, psq_hbm.at[wid])
        pltpu.sync_copy(cnt_v, pcnt_hbm.at[wid])

    return k(features, patient_ids)


def _epi_body(ps_ref, sq_ref, pc_ref, out_ref):
    sums = ps_ref[0]
    cnt_l = pc_ref[0]
    for t in range(1, NW):
        sums = sums + ps_ref[t]
        cnt_l = cnt_l + pc_ref[t]
    total_sq = jnp.sum(sq_ref[...])
    cnt = jnp.broadcast_to(cnt_l[:, 0:1], (NSEG, D))

    safe = jnp.maximum(cnt, 1.0)
    cent = sums / safe
    csq = cent * cent
    within = total_sq - jnp.sum(cnt * csq)
    validf = (cnt > 0).astype(jnp.float32)
    kseg = jnp.sum(validf) / D
    csqsum = jnp.sum(validf * csq)
    svec = jnp.sum(validf * cent, axis=0, keepdims=True)
    ssq = jnp.sum(svec * svec)
    between = kseg * csqsum - ssq
    loss_pcsl = within / (between + EPS)
    loss_gpal = (csqsum - ssq / kseg) / kseg
    loss = LAMBDA_PCSL * loss_pcsl + LAMBDA_GPAL * loss_gpal
    out_ref[...] = jnp.broadcast_to(loss, (1, 1))


def kernel(features, patient_ids):
    psum, psq, pcnt = _sc_partials(features, patient_ids)
    out = pl.pallas_call(
        _epi_body,
        out_shape=jax.ShapeDtypeStruct((1, 1), jnp.float32),
    )(psum, psq, pcnt)
    return out[0, 0]
